# baseline (speedup 1.0000x reference)
import functools
import numpy as np
import jax
import jax.numpy as jnp
from jax import lax
from jax.experimental import pallas as pl
from jax.experimental.pallas import tpu as pltpu
from jax.experimental.pallas import tpu_sc as plsc

F = 128
BLK = 2000
L = 16
try:
    _info = plsc.get_sparse_core_info()
    NC, NS = _info.num_cores, _info.num_subcores
except Exception:
    NC, NS = 2, 16
NW = NC * NS
CHUNK = 3136
NPAD = NW * CHUNK


def _softplus(w):
    return jnp.maximum(w, 0.0) + jnp.log(1.0 + jnp.exp(-jnp.abs(w)))


def _swish(u):
    return u * (1.0 / (1.0 + jnp.exp(-u)))


def _dot(a, b, dims):
    return jax.lax.dot_general(a, b, (dims, ((), ())),
                               preferred_element_type=jnp.float32)


def _body_a(seg_ref, x_ref, esc_ref, wqk_ref, bqk_ref, a_ref, *, nseg):
    x = x_ref[...]
    seg = seg_ref[0]
    t = _dot(wqk_ref[...], x, ((1,), (1,))) + bqk_ref[0]
    ids = lax.broadcasted_iota(jnp.int32, (nseg, BLK), 0)
    oh = (ids == seg).astype(jnp.float32)
    esc_g = _dot(esc_ref[...], oh, ((1,), (0,)))
    a_ref[...] = _softplus(esc_g * t).reshape(1, 1, BLK)


def _sc_b1(a_hbm, seg_hbm, pan_hbm, a_v, seg_v, pan_v, *, nseg):
    wid = lax.axis_index("s") * NC + lax.axis_index("c")
    base = wid * CHUNK
    pltpu.sync_copy(a_hbm.at[pl.ds(base, CHUNK)], a_v)
    pltpu.sync_copy(seg_hbm.at[pl.ds(base, CHUNK)], seg_v)
    for k in range(nseg // L):
        pan_v[pl.ds(k * L, L)] = jnp.zeros((L,), jnp.float32)

    def body(j, c):
        idx = seg_v[pl.ds(j * L, L)]
        av = a_v[pl.ds(j * L, L)]
        plsc.addupdate_scatter(pan_v, [idx], av)
        return c

    lax.fori_loop(0, CHUNK // L, body, 0)
    pltpu.sync_copy(pan_v, pan_hbm.at[pl.ds(wid * nseg, nseg)])


def _sc_b2(a_hbm, seg_hbm, pan_hbm, etab_hbm, coeff_hbm,
           a_v, seg_v, pan_v, anorm_v, etab_v, coeff_v, *, nseg):
    wid = lax.axis_index("s") * NC + lax.axis_index("c")
    base = wid * CHUNK
    pltpu.sync_copy(a_hbm.at[pl.ds(base, CHUNK)], a_v)
    pltpu.sync_copy(seg_hbm.at[pl.ds(base, CHUNK)], seg_v)
    pltpu.sync_copy(pan_hbm, pan_v)
    pltpu.sync_copy(etab_hbm, etab_v)
    for k in range(nseg // L):
        def rbody(r, acc):
            return acc + pan_v[pl.ds(r * nseg + k * L, L)]
        anorm_v[pl.ds(k * L, L)] = lax.fori_loop(
            0, NW, rbody, jnp.zeros((L,), jnp.float32))

    def body(j, c):
        idx = seg_v[pl.ds(j * L, L)]
        av = a_v[pl.ds(j * L, L)]
        ag = plsc.load_gather(anorm_v, [idx])
        eg = plsc.load_gather(etab_v, [idx])
        coeff_v[pl.ds(j * L, L)] = av / (ag + 1e-8) * eg
        return c

    lax.fori_loop(0, CHUNK // L, body, 0)
    pltpu.sync_copy(coeff_v, coeff_hbm.at[pl.ds(base, CHUNK)])


def _body_c(coeff_ref, wv_ref, w1_ref, w2_ref, wl_ref, out_ref):
    coeff = coeff_ref[0]
    scaled = _dot(coeff, wv_ref[...], ((0,), (0,)))
    s1 = _swish(scaled)
    u = _dot(s1, w1_ref[...], ((1,), (1,)))
    s2 = _swish(u)
    h = scaled + _dot(s2, w2_ref[...], ((1,), (1,)))
    out_ref[...] = _dot(_swish(h), wl_ref[...], ((1,), (1,)))


def kernel(x, E, num_batch, batch_seg, Wq, bq, Wk, Wv, W1, W2, Wl):
    N = x.shape[0]
    nseg = E.shape[0]
    nblk = N // BLK
    inv = np.float32(1.0 / np.sqrt(F))

    wqk = (Wq.T @ Wk).reshape(1, F) * inv
    bqk = (bq @ Wk).reshape(1, 1) * inv
    e = jnp.abs(E)
    esc = (e / jnp.maximum(e, 1.0)).reshape(1, nseg)
    seg3 = batch_seg.reshape(nblk, 1, BLK)
    wv = Wv.reshape(1, F)

    a3 = pl.pallas_call(
        functools.partial(_body_a, nseg=nseg),
        grid=(nblk,),
        in_specs=[
            pl.BlockSpec((1, 1, BLK), lambda i: (i, 0, 0)),
            pl.BlockSpec((BLK, F), lambda i: (i, 0)),
            pl.BlockSpec((1, nseg), lambda i: (0, 0)),
            pl.BlockSpec((1, F), lambda i: (0, 0)),
            pl.BlockSpec((1, 1), lambda i: (0, 0)),
        ],
        out_specs=pl.BlockSpec((1, 1, BLK), lambda i: (i, 0, 0)),
        out_shape=jax.ShapeDtypeStruct((nblk, 1, BLK), jnp.float32),
    )(seg3, x, esc, wqk, bqk)

    a_pad = jnp.pad(a3.reshape(N), (0, NPAD - N))
    seg_pad = jnp.pad(batch_seg, (0, NPAD - N))

    mesh = plsc.VectorSubcoreMesh(core_axis_name="c", subcore_axis_name="s")

    pan = pl.kernel(
        functools.partial(_sc_b1, nseg=nseg),
        out_type=jax.ShapeDtypeStruct((NW * nseg,), jnp.float32),
        mesh=mesh,
        compiler_params=pltpu.CompilerParams(needs_layout_passes=False),
        scratch_types=[
            pltpu.VMEM((CHUNK,), jnp.float32),
            pltpu.VMEM((CHUNK,), jnp.int32),
            pltpu.VMEM((nseg,), jnp.float32),
        ],
    )(a_pad, seg_pad)

    coeff = pl.kernel(
        functools.partial(_sc_b2, nseg=nseg),
        out_type=jax.ShapeDtypeStruct((NPAD,), jnp.float32),
        mesh=mesh,
        compiler_params=pltpu.CompilerParams(needs_layout_passes=False),
        scratch_types=[
            pltpu.VMEM((CHUNK,), jnp.float32),
            pltpu.VMEM((CHUNK,), jnp.int32),
            pltpu.VMEM((NW * nseg,), jnp.float32),
            pltpu.VMEM((nseg,), jnp.float32),
            pltpu.VMEM((nseg,), jnp.float32),
            pltpu.VMEM((CHUNK,), jnp.float32),
        ],
    )(a_pad, seg_pad, pan, e)

    coeff3 = coeff[:N].reshape(nblk, 1, BLK)

    out = pl.pallas_call(
        _body_c,
        grid=(nblk,),
        in_specs=[
            pl.BlockSpec((1, 1, BLK), lambda i: (i, 0, 0)),
            pl.BlockSpec((1, F), lambda i: (0, 0)),
            pl.BlockSpec((F, F), lambda i: (0, 0)),
            pl.BlockSpec((F, F), lambda i: (0, 0)),
            pl.BlockSpec((F, F), lambda i: (0, 0)),
        ],
        out_specs=pl.BlockSpec((BLK, F), lambda i: (i, 0)),
        out_shape=jax.ShapeDtypeStruct((N, F), jnp.float32),
    )(coeff3, wv, W1, W2, Wl)
    return out
